# Initial kernel scaffold; baseline (speedup 1.0000x reference)
import jax
import jax.numpy as jnp
from jax.experimental import pallas as pl


def kernel(x, e, edge_index, W7, b7, W8, b8, W81, b81, W9, b9):
    raise NotImplementedError("write your pallas kernel here")


# speedup vs baseline: 2.0313x; 2.0313x over previous
import functools

import jax
import jax.numpy as jnp
from jax import lax
from jax.experimental import pallas as pl
from jax.experimental.pallas import tpu as pltpu
from jax.experimental.pallas import tpu_sc as plsc

N_NODES = 10000
N_EDGES = 320000
D_NODE = 128
D_EDGE = 16

NC = 2
NS = 16
NW = NC * NS
EDGES_PER_W = N_EDGES // NW
CHUNK = 80
CHUNKS_PER_W = EDGES_PER_W // CHUNK
GROUPS = CHUNK // 16

_W8_OFF = 0
_B8_OFF = 72
_W81_OFF = 84
_B81_OFF = 156
_W9_OFF = 162
_B9_OFF = 174
_WTAB_ROWS = 176


def _lrelu(v):
    return jnp.where(v >= 0.0, v, v * 0.1)


def _node_proj_body(x_ref, ws_ref, wd_ref, zs_ref, zd_ref):
    xb = x_ref[...]
    zs_ref[...] = lax.dot_general(
        xb, ws_ref[...], (((1,), (0,)), ((), ())),
        preferred_element_type=jnp.float32, precision=lax.Precision.HIGHEST)
    zd_ref[...] = lax.dot_general(
        xb, wd_ref[...], (((1,), (0,)), ((), ())),
        preferred_element_type=jnp.float32, precision=lax.Precision.HIGHEST)


def _node_proj(x, ws_pad, wd_pad):
    blk = 1000
    grid = N_NODES // blk
    return pl.pallas_call(
        _node_proj_body,
        grid=(grid,),
        in_specs=[
            pl.BlockSpec((blk, D_NODE), lambda i: (i, 0)),
            pl.BlockSpec((D_NODE, 16), lambda i: (0, 0)),
            pl.BlockSpec((D_NODE, 16), lambda i: (0, 0)),
        ],
        out_specs=[
            pl.BlockSpec((blk, 16), lambda i: (i, 0)),
            pl.BlockSpec((blk, 16), lambda i: (i, 0)),
        ],
        out_shape=[
            jax.ShapeDtypeStruct((N_NODES, 16), jnp.float32),
            jax.ShapeDtypeStruct((N_NODES, 16), jnp.float32),
        ],
    )(x, ws_pad, wd_pad)


def _edge_proj_body(e_ref, w_ref, b_ref, z_ref):
    z = lax.dot_general(
        e_ref[...], w_ref[...], (((1,), (0,)), ((), ())),
        preferred_element_type=jnp.float32, precision=lax.Precision.HIGHEST)
    z_ref[...] = z + b_ref[...]


def _edge_proj(e, we_pad, b7_pad):
    blk = 8000
    grid = N_EDGES // blk
    return pl.pallas_call(
        _edge_proj_body,
        grid=(grid,),
        in_specs=[
            pl.BlockSpec((blk, D_EDGE), lambda i: (i, 0)),
            pl.BlockSpec((D_EDGE, 8), lambda i: (0, 0)),
            pl.BlockSpec((1, 8), lambda i: (0, 0)),
        ],
        out_specs=pl.BlockSpec((blk, 8), lambda i: (i, 0)),
        out_shape=jax.ShapeDtypeStruct((N_EDGES, 8), jnp.float32),
    )(e, we_pad, b7_pad)


def _sc_body(zs_hbm, zd_hbm, ze_hbm, src_hbm, dst_hbm, wtab_hbm, out_hbm,
             idx_s, idx_d, buf_a, buf_b, ze_buf, out_buf, wbuf, sem):
    wid = lax.axis_index("s") * NC + lax.axis_index("c")
    pltpu.sync_copy(wtab_hbm, wbuf)

    iota16 = lax.iota(jnp.int32, 16)
    zeros16 = jnp.zeros((16,), jnp.int32)
    ones16 = jnp.full((16,), 1, jnp.int32)
    col_idx = [jnp.full((16,), k, jnp.int32) for k in range(6)]

    def chunk_body(j, carry):
        gid = wid * CHUNKS_PER_W + j
        r0 = gid * CHUNK
        pltpu.sync_copy(src_hbm.at[gid], idx_s)
        pltpu.sync_copy(dst_hbm.at[gid], idx_d)
        cp_a = pltpu.async_copy(zs_hbm.at[idx_s], buf_a, sem)
        cp_b = pltpu.async_copy(zd_hbm.at[idx_d], buf_b, sem)
        pltpu.sync_copy(ze_hbm.at[pl.ds(r0, CHUNK)], ze_buf)
        cp_a.wait()
        cp_b.wait()

        def group_body(g, gcarry):
            rows = g * 16 + iota16
            h0 = []
            for k in range(6):
                a = plsc.load_gather(buf_a, [rows, col_idx[k]])
                b = plsc.load_gather(buf_b, [rows, col_idx[k]])
                z = plsc.load_gather(ze_buf, [rows, col_idx[k]])
                h0.append(_lrelu(a + b + z))
            h1 = []
            for jj in range(12):
                acc = wbuf[_B8_OFF + jj]
                for k in range(6):
                    acc = acc + h0[k] * wbuf[_W8_OFF + k * 12 + jj]
                h1.append(_lrelu(acc))
            h2 = []
            for jj in range(6):
                acc = wbuf[_B81_OFF + jj]
                for k in range(12):
                    acc = acc + h1[k] * wbuf[_W81_OFF + k * 6 + jj]
                h2.append(_lrelu(acc))
            o = []
            for jj in range(2):
                acc = wbuf[_B9_OFF + jj]
                for k in range(6):
                    acc = acc + h2[k] * wbuf[_W9_OFF + k * 2 + jj]
                o.append(acc)
            m = jnp.maximum(o[0], o[1])
            e0 = jnp.exp(o[0] - m)
            e1 = jnp.exp(o[1] - m)
            inv = 1.0 / (e0 + e1)
            plsc.store_scatter(out_buf, [rows, zeros16], e0 * inv)
            plsc.store_scatter(out_buf, [rows, ones16], e1 * inv)
            return gcarry

        lax.fori_loop(0, GROUPS, group_body, 0)
        pltpu.sync_copy(out_buf, out_hbm.at[pl.ds(r0, CHUNK)])
        return carry

    lax.fori_loop(0, CHUNKS_PER_W, chunk_body, 0)


def _sc_edge_mlp(zs, zd, ze, src2d, dst2d, wtab):
    mesh = plsc.VectorSubcoreMesh(core_axis_name="c", subcore_axis_name="s")
    fn = functools.partial(
        pl.kernel,
        out_type=jax.ShapeDtypeStruct((N_EDGES, 2), jnp.float32),
        mesh=mesh,
        compiler_params=pltpu.CompilerParams(
            needs_layout_passes=False, use_tc_tiling_on_sc=False),
        scratch_types=[
            pltpu.VMEM((CHUNK,), jnp.int32),
            pltpu.VMEM((CHUNK,), jnp.int32),
            pltpu.VMEM((CHUNK, 16), jnp.float32),
            pltpu.VMEM((CHUNK, 16), jnp.float32),
            pltpu.VMEM((CHUNK, 8), jnp.float32),
            pltpu.VMEM((CHUNK, 2), jnp.float32),
            pltpu.VMEM((_WTAB_ROWS, 16), jnp.float32),
            pltpu.SemaphoreType.DMA,
        ],
    )(_sc_body)
    return fn(zs, zd, ze, src2d, dst2d, wtab)


def kernel(x, e, edge_index, W7, b7, W8, b8, W81, b81, W9, b9):
    src2d = edge_index[0].astype(jnp.int32).reshape(N_EDGES // CHUNK, CHUNK)
    dst2d = edge_index[1].astype(jnp.int32).reshape(N_EDGES // CHUNK, CHUNK)

    ws_pad = jnp.zeros((D_NODE, 16), jnp.float32).at[:, :6].set(W7[:D_NODE])
    wd_pad = jnp.zeros((D_NODE, 16), jnp.float32).at[:, :6].set(
        W7[D_NODE:2 * D_NODE])
    we_pad = jnp.zeros((D_EDGE, 8), jnp.float32).at[:, :6].set(
        W7[2 * D_NODE:])
    b7_pad = jnp.zeros((1, 8), jnp.float32).at[0, :6].set(b7)

    zs, zd = _node_proj(x, ws_pad, wd_pad)
    ze = _edge_proj(e, we_pad, b7_pad)

    wtab = jnp.concatenate([
        W8.reshape(72), b8, W81.reshape(72), b81, W9.reshape(12), b9])
    wtab = jnp.broadcast_to(wtab[:, None], (_WTAB_ROWS, 16))

    return _sc_edge_mlp(zs, zd, ze, src2d, dst2d, wtab)


# speedup vs baseline: 2.8568x; 1.4064x over previous
import functools

import jax
import jax.numpy as jnp
from jax import lax
from jax.experimental import pallas as pl
from jax.experimental.pallas import tpu as pltpu
from jax.experimental.pallas import tpu_sc as plsc

N_NODES = 10000
N_EDGES = 320000
D_NODE = 128
D_EDGE = 16

NC = 2
NS = 16
NW = NC * NS
EDGES_PER_W = N_EDGES // NW
SUB = 80
SUBS = 5
CHUNK = SUB * SUBS
CHUNKS_PER_W = EDGES_PER_W // CHUNK
GROUPS = CHUNK // 16

_W8_OFF = 0
_B8_OFF = 72
_W81_OFF = 84
_B81_OFF = 156
_W9_OFF = 162
_B9_OFF = 174
_WTAB_ROWS = 176


def _lrelu(v):
    return jnp.where(v >= 0.0, v, v * 0.1)


def _node_proj_body(x_ref, ws_ref, wd_ref, zs_ref, zd_ref):
    xb = x_ref[...]
    zs_ref[...] = lax.dot_general(
        xb, ws_ref[...], (((1,), (0,)), ((), ())),
        preferred_element_type=jnp.float32, precision=lax.Precision.HIGHEST)
    zd_ref[...] = lax.dot_general(
        xb, wd_ref[...], (((1,), (0,)), ((), ())),
        preferred_element_type=jnp.float32, precision=lax.Precision.HIGHEST)


def _node_proj(x, ws_pad, wd_pad):
    blk = 1000
    grid = N_NODES // blk
    return pl.pallas_call(
        _node_proj_body,
        grid=(grid,),
        in_specs=[
            pl.BlockSpec((blk, D_NODE), lambda i: (i, 0)),
            pl.BlockSpec((D_NODE, 16), lambda i: (0, 0)),
            pl.BlockSpec((D_NODE, 16), lambda i: (0, 0)),
        ],
        out_specs=[
            pl.BlockSpec((blk, 16), lambda i: (i, 0)),
            pl.BlockSpec((blk, 16), lambda i: (i, 0)),
        ],
        out_shape=[
            jax.ShapeDtypeStruct((N_NODES, 16), jnp.float32),
            jax.ShapeDtypeStruct((N_NODES, 16), jnp.float32),
        ],
    )(x, ws_pad, wd_pad)


def _edge_proj_body(e_ref, wt_ref, bt_ref, z_ref):
    z = lax.dot_general(
        wt_ref[...], e_ref[...], (((1,), (1,)), ((), ())),
        preferred_element_type=jnp.float32, precision=lax.Precision.HIGHEST)
    z_ref[...] = z + bt_ref[...]


def _edge_proj_t(e, wet_pad, b7t_pad):
    blk = 12800
    grid = N_EDGES // blk
    return pl.pallas_call(
        _edge_proj_body,
        grid=(grid,),
        in_specs=[
            pl.BlockSpec((blk, D_EDGE), lambda i: (i, 0)),
            pl.BlockSpec((8, D_EDGE), lambda i: (0, 0)),
            pl.BlockSpec((8, 1), lambda i: (0, 0)),
        ],
        out_specs=pl.BlockSpec((8, blk), lambda i: (0, i)),
        out_shape=jax.ShapeDtypeStruct((8, N_EDGES), jnp.float32),
    )(e, wet_pad, b7t_pad)


def _sc_body(zs_hbm, zd_hbm, zet_hbm, src_hbm, dst_hbm, wtab_hbm, out_hbm,
             idx_s, idx_d, buf_a, buf_b, ze_buf, out_buf, wbuf, sem):
    wid = lax.axis_index("s") * NC + lax.axis_index("c")
    pltpu.sync_copy(wtab_hbm, wbuf)

    iota16 = lax.iota(jnp.int32, 16)
    zeros16 = jnp.zeros((16,), jnp.int32)
    ones16 = jnp.full((16,), 1, jnp.int32)
    col_idx = [jnp.full((16,), k, jnp.int32) for k in range(6)]

    def chunk_body(j, carry):
        gid = wid * CHUNKS_PER_W + j
        r0 = gid * CHUNK
        pltpu.sync_copy(src_hbm.at[pl.ds(gid * SUBS, SUBS)], idx_s)
        pltpu.sync_copy(dst_hbm.at[pl.ds(gid * SUBS, SUBS)], idx_d)
        copies = []
        for k in range(SUBS):
            copies.append(pltpu.async_copy(
                zs_hbm.at[idx_s.at[k]], buf_a.at[pl.ds(k * SUB, SUB)], sem))
            copies.append(pltpu.async_copy(
                zd_hbm.at[idx_d.at[k]], buf_b.at[pl.ds(k * SUB, SUB)], sem))
        pltpu.sync_copy(zet_hbm.at[:, pl.ds(r0, CHUNK)], ze_buf)
        for cp in copies:
            cp.wait()

        def group_body(g, gcarry):
            rows = g * 16 + iota16
            g16 = g * 16
            h0 = []
            for k in range(6):
                a = plsc.load_gather(buf_a, [rows, col_idx[k]])
                b = plsc.load_gather(buf_b, [rows, col_idx[k]])
                z = ze_buf[k, pl.ds(g16, 16)]
                h0.append(_lrelu(a + b + z))
            h1 = []
            for jj in range(12):
                acc = wbuf[_B8_OFF + jj]
                for k in range(6):
                    acc = acc + h0[k] * wbuf[_W8_OFF + k * 12 + jj]
                h1.append(_lrelu(acc))
            h2 = []
            for jj in range(6):
                acc = wbuf[_B81_OFF + jj]
                for k in range(12):
                    acc = acc + h1[k] * wbuf[_W81_OFF + k * 6 + jj]
                h2.append(_lrelu(acc))
            o = []
            for jj in range(2):
                acc = wbuf[_B9_OFF + jj]
                for k in range(6):
                    acc = acc + h2[k] * wbuf[_W9_OFF + k * 2 + jj]
                o.append(acc)
            m = jnp.maximum(o[0], o[1])
            e0 = jnp.exp(o[0] - m)
            e1 = jnp.exp(o[1] - m)
            inv = 1.0 / (e0 + e1)
            plsc.store_scatter(out_buf, [rows, zeros16], e0 * inv)
            plsc.store_scatter(out_buf, [rows, ones16], e1 * inv)
            return gcarry

        lax.fori_loop(0, GROUPS, group_body, 0)
        pltpu.sync_copy(out_buf, out_hbm.at[pl.ds(r0, CHUNK)])
        return carry

    lax.fori_loop(0, CHUNKS_PER_W, chunk_body, 0)


def _sc_edge_mlp(zs, zd, zet, src2d, dst2d, wtab):
    mesh = plsc.VectorSubcoreMesh(core_axis_name="c", subcore_axis_name="s")
    fn = functools.partial(
        pl.kernel,
        out_type=jax.ShapeDtypeStruct((N_EDGES, 2), jnp.float32),
        mesh=mesh,
        compiler_params=pltpu.CompilerParams(
            needs_layout_passes=False, use_tc_tiling_on_sc=False),
        scratch_types=[
            pltpu.VMEM((SUBS, SUB), jnp.int32),
            pltpu.VMEM((SUBS, SUB), jnp.int32),
            pltpu.VMEM((CHUNK, 16), jnp.float32),
            pltpu.VMEM((CHUNK, 16), jnp.float32),
            pltpu.VMEM((8, CHUNK), jnp.float32),
            pltpu.VMEM((CHUNK, 2), jnp.float32),
            pltpu.VMEM((_WTAB_ROWS, 16), jnp.float32),
            pltpu.SemaphoreType.DMA,
        ],
    )(_sc_body)
    return fn(zs, zd, zet, src2d, dst2d, wtab)


def kernel(x, e, edge_index, W7, b7, W8, b8, W81, b81, W9, b9):
    src2d = edge_index[0].astype(jnp.int32).reshape(N_EDGES // SUB, SUB)
    dst2d = edge_index[1].astype(jnp.int32).reshape(N_EDGES // SUB, SUB)

    ws_pad = jnp.zeros((D_NODE, 16), jnp.float32).at[:, :6].set(W7[:D_NODE])
    wd_pad = jnp.zeros((D_NODE, 16), jnp.float32).at[:, :6].set(
        W7[D_NODE:2 * D_NODE])
    wet_pad = jnp.zeros((8, D_EDGE), jnp.float32).at[:6, :].set(
        W7[2 * D_NODE:].T)
    b7t_pad = jnp.zeros((8, 1), jnp.float32).at[:6, 0].set(b7)

    zs, zd = _node_proj(x, ws_pad, wd_pad)
    zet = _edge_proj_t(e, wet_pad, b7t_pad)

    wtab = jnp.concatenate([
        W8.reshape(72), b8, W81.reshape(72), b81, W9.reshape(12), b9])
    wtab = jnp.broadcast_to(wtab[:, None], (_WTAB_ROWS, 16))

    return _sc_edge_mlp(zs, zd, zet, src2d, dst2d, wtab)


# speedup vs baseline: 3.0505x; 1.0678x over previous
import functools

import jax
import jax.numpy as jnp
from jax import lax
from jax.experimental import pallas as pl
from jax.experimental.pallas import tpu as pltpu
from jax.experimental.pallas import tpu_sc as plsc

N_NODES = 10000
N_EDGES = 320000
D_NODE = 128
D_EDGE = 16

NC = 2
NS = 16
NW = NC * NS
EDGES_PER_W = N_EDGES // NW
SUB = 80
SUBS = 5
CHUNK = SUB * SUBS
CHUNKS_PER_W = EDGES_PER_W // CHUNK
GROUPS = CHUNK // 16

_W8_OFF = 0
_B8_OFF = 72
_W81_OFF = 84
_B81_OFF = 156
_W9_OFF = 162
_B9_OFF = 174
_WTAB_ROWS = 176


def _lrelu(v):
    return jnp.where(v >= 0.0, v, v * 0.1)


def _node_proj_body(x_ref, ws_ref, wd_ref, zs_ref, zd_ref):
    xb = x_ref[...]
    zs_ref[...] = lax.dot_general(
        xb, ws_ref[...], (((1,), (0,)), ((), ())),
        preferred_element_type=jnp.float32, precision=lax.Precision.HIGHEST)
    zd_ref[...] = lax.dot_general(
        xb, wd_ref[...], (((1,), (0,)), ((), ())),
        preferred_element_type=jnp.float32, precision=lax.Precision.HIGHEST)


def _node_proj(x, ws_pad, wd_pad):
    blk = 1000
    grid = N_NODES // blk
    return pl.pallas_call(
        _node_proj_body,
        grid=(grid,),
        in_specs=[
            pl.BlockSpec((blk, D_NODE), lambda i: (i, 0)),
            pl.BlockSpec((D_NODE, 16), lambda i: (0, 0)),
            pl.BlockSpec((D_NODE, 16), lambda i: (0, 0)),
        ],
        out_specs=[
            pl.BlockSpec((blk, 16), lambda i: (i, 0)),
            pl.BlockSpec((blk, 16), lambda i: (i, 0)),
        ],
        out_shape=[
            jax.ShapeDtypeStruct((N_NODES, 16), jnp.float32),
            jax.ShapeDtypeStruct((N_NODES, 16), jnp.float32),
        ],
    )(x, ws_pad, wd_pad)


def _edge_proj_body(e_ref, wt_ref, bt_ref, z_ref):
    z = lax.dot_general(
        wt_ref[...], e_ref[...], (((1,), (1,)), ((), ())),
        preferred_element_type=jnp.float32, precision=lax.Precision.HIGHEST)
    z_ref[...] = z + bt_ref[...]


def _edge_proj_t(e, wet_pad, b7t_pad):
    blk = 12800
    grid = N_EDGES // blk
    return pl.pallas_call(
        _edge_proj_body,
        grid=(grid,),
        in_specs=[
            pl.BlockSpec((blk, D_EDGE), lambda i: (i, 0)),
            pl.BlockSpec((8, D_EDGE), lambda i: (0, 0)),
            pl.BlockSpec((8, 1), lambda i: (0, 0)),
        ],
        out_specs=pl.BlockSpec((8, blk), lambda i: (0, i)),
        out_shape=jax.ShapeDtypeStruct((8, N_EDGES), jnp.float32),
    )(e, wet_pad, b7t_pad)


def _sc_body(zs_hbm, zd_hbm, zet_hbm, src_hbm, dst_hbm, wtab_hbm, out_hbm,
             idx_s, idx_d, buf_a, buf_b, ze_buf, out_buf, wbuf,
             sem_g, sem_o):
    wid = lax.axis_index("s") * NC + lax.axis_index("c")
    pltpu.sync_copy(wtab_hbm, wbuf)

    iota16 = lax.iota(jnp.int32, 16)
    zeros16 = jnp.zeros((16,), jnp.int32)
    ones16 = jnp.full((16,), 1, jnp.int32)
    col_idx = [jnp.full((16,), k, jnp.int32) for k in range(6)]

    def start_fetch(j, par):
        gid = wid * CHUNKS_PER_W + j
        r0 = gid * CHUNK
        pltpu.sync_copy(src_hbm.at[pl.ds(gid * SUBS, SUBS)], idx_s.at[par])
        pltpu.sync_copy(dst_hbm.at[pl.ds(gid * SUBS, SUBS)], idx_d.at[par])
        for k in range(SUBS):
            pltpu.async_copy(
                zs_hbm.at[idx_s.at[par].at[k]],
                buf_a.at[par].at[pl.ds(k * SUB, SUB)], sem_g.at[par])
            pltpu.async_copy(
                zd_hbm.at[idx_d.at[par].at[k]],
                buf_b.at[par].at[pl.ds(k * SUB, SUB)], sem_g.at[par])
        pltpu.async_copy(
            zet_hbm.at[:, pl.ds(r0, CHUNK)], ze_buf.at[par], sem_g.at[par])

    def drain_fetch(par):
        for k in range(SUBS):
            pltpu.make_async_copy(
                zs_hbm.at[idx_s.at[par].at[k]],
                buf_a.at[par].at[pl.ds(k * SUB, SUB)], sem_g.at[par]).wait()
            pltpu.make_async_copy(
                zd_hbm.at[idx_d.at[par].at[k]],
                buf_b.at[par].at[pl.ds(k * SUB, SUB)], sem_g.at[par]).wait()
        pltpu.make_async_copy(
            zet_hbm.at[:, pl.ds(0, CHUNK)], ze_buf.at[par],
            sem_g.at[par]).wait()

    def drain_out(j, par):
        r0 = (wid * CHUNKS_PER_W + j) * CHUNK
        pltpu.make_async_copy(
            out_buf.at[par], out_hbm.at[pl.ds(r0, CHUNK)], sem_o.at[par]
        ).wait()

    start_fetch(0, 0)

    def chunk_body(j, carry):
        par = lax.rem(j, 2)
        nxt = lax.rem(j + 1, 2)
        gid = wid * CHUNKS_PER_W + j
        r0 = gid * CHUNK

        @pl.when(j + 1 < CHUNKS_PER_W)
        def _():
            start_fetch(j + 1, nxt)

        drain_fetch(par)

        bfa = buf_a.at[par]
        bfb = buf_b.at[par]
        zeb = ze_buf.at[par]
        obf = out_buf.at[par]

        def group_body(g, gcarry):
            rows = g * 16 + iota16
            g16 = g * 16
            h0 = []
            for k in range(6):
                a = plsc.load_gather(bfa, [rows, col_idx[k]])
                b = plsc.load_gather(bfb, [rows, col_idx[k]])
                z = zeb[k, pl.ds(g16, 16)]
                h0.append(_lrelu(a + b + z))
            h1 = []
            for jj in range(12):
                acc = wbuf[_B8_OFF + jj]
                for k in range(6):
                    acc = acc + h0[k] * wbuf[_W8_OFF + k * 12 + jj]
                h1.append(_lrelu(acc))
            h2 = []
            for jj in range(6):
                acc = wbuf[_B81_OFF + jj]
                for k in range(12):
                    acc = acc + h1[k] * wbuf[_W81_OFF + k * 6 + jj]
                h2.append(_lrelu(acc))
            o = []
            for jj in range(2):
                acc = wbuf[_B9_OFF + jj]
                for k in range(6):
                    acc = acc + h2[k] * wbuf[_W9_OFF + k * 2 + jj]
                o.append(acc)
            m = jnp.maximum(o[0], o[1])
            e0 = jnp.exp(o[0] - m)
            e1 = jnp.exp(o[1] - m)
            inv = 1.0 / (e0 + e1)
            plsc.store_scatter(obf, [rows, zeros16], e0 * inv)
            plsc.store_scatter(obf, [rows, ones16], e1 * inv)
            return gcarry

        lax.fori_loop(0, GROUPS, group_body, 0)

        @pl.when(j >= 2)
        def _():
            drain_out(j - 2, par)

        pltpu.async_copy(obf, out_hbm.at[pl.ds(r0, CHUNK)], sem_o.at[par])
        return carry

    lax.fori_loop(0, CHUNKS_PER_W, chunk_body, 0)
    drain_out(CHUNKS_PER_W - 2, lax.rem(CHUNKS_PER_W - 2, 2))
    drain_out(CHUNKS_PER_W - 1, lax.rem(CHUNKS_PER_W - 1, 2))


def _sc_edge_mlp(zs, zd, zet, src2d, dst2d, wtab):
    mesh = plsc.VectorSubcoreMesh(core_axis_name="c", subcore_axis_name="s")
    fn = functools.partial(
        pl.kernel,
        out_type=jax.ShapeDtypeStruct((N_EDGES, 2), jnp.float32),
        mesh=mesh,
        compiler_params=pltpu.CompilerParams(
            needs_layout_passes=False, use_tc_tiling_on_sc=False),
        scratch_types=[
            pltpu.VMEM((2, SUBS, SUB), jnp.int32),
            pltpu.VMEM((2, SUBS, SUB), jnp.int32),
            pltpu.VMEM((2, CHUNK, 16), jnp.float32),
            pltpu.VMEM((2, CHUNK, 16), jnp.float32),
            pltpu.VMEM((2, 8, CHUNK), jnp.float32),
            pltpu.VMEM((2, CHUNK, 2), jnp.float32),
            pltpu.VMEM((_WTAB_ROWS, 16), jnp.float32),
            pltpu.SemaphoreType.DMA((2,)),
            pltpu.SemaphoreType.DMA((2,)),
        ],
    )(_sc_body)
    return fn(zs, zd, zet, src2d, dst2d, wtab)


def kernel(x, e, edge_index, W7, b7, W8, b8, W81, b81, W9, b9):
    src2d = edge_index[0].astype(jnp.int32).reshape(N_EDGES // SUB, SUB)
    dst2d = edge_index[1].astype(jnp.int32).reshape(N_EDGES // SUB, SUB)

    ws_pad = jnp.zeros((D_NODE, 16), jnp.float32).at[:, :6].set(W7[:D_NODE])
    wd_pad = jnp.zeros((D_NODE, 16), jnp.float32).at[:, :6].set(
        W7[D_NODE:2 * D_NODE])
    wet_pad = jnp.zeros((8, D_EDGE), jnp.float32).at[:6, :].set(
        W7[2 * D_NODE:].T)
    b7t_pad = jnp.zeros((8, 1), jnp.float32).at[:6, 0].set(b7)

    zs, zd = _node_proj(x, ws_pad, wd_pad)
    zet = _edge_proj_t(e, wet_pad, b7t_pad)

    wtab = jnp.concatenate([
        W8.reshape(72), b8, W81.reshape(72), b81, W9.reshape(12), b9])
    wtab = jnp.broadcast_to(wtab[:, None], (_WTAB_ROWS, 16))

    return _sc_edge_mlp(zs, zd, zet, src2d, dst2d, wtab)


# speedup vs baseline: 7.3672x; 2.4151x over previous
import functools

import jax
import jax.numpy as jnp
from jax import lax
from jax.experimental import pallas as pl
from jax.experimental.pallas import tpu as pltpu
from jax.experimental.pallas import tpu_sc as plsc

N_NODES = 10000
N_EDGES = 320000
D_NODE = 128
D_EDGE = 16

NC = 2
NS = 16
NW = NC * NS
EDGES_PER_W = N_EDGES // NW
SUB = 80
SUBS = 5
CHUNK = SUB * SUBS
CHUNKS_PER_W = EDGES_PER_W // CHUNK
GROUPS = CHUNK // 16

_W8_OFF = 0
_B8_OFF = 72
_W81_OFF = 84
_B81_OFF = 156
_W9_OFF = 162
_B9_OFF = 174
_WTAB_ROWS = 176


def _lrelu(v):
    return jnp.where(v >= 0.0, v, v * 0.1)


def _node_proj_body(x_ref, ws_ref, wd_ref, zs_ref, zd_ref):
    xb = x_ref[...]
    zs_ref[...] = lax.dot_general(
        xb, ws_ref[...], (((1,), (0,)), ((), ())),
        preferred_element_type=jnp.float32, precision=lax.Precision.HIGHEST)
    zd_ref[...] = lax.dot_general(
        xb, wd_ref[...], (((1,), (0,)), ((), ())),
        preferred_element_type=jnp.float32, precision=lax.Precision.HIGHEST)


def _node_proj(x, ws_pad, wd_pad):
    blk = 1000
    grid = N_NODES // blk
    return pl.pallas_call(
        _node_proj_body,
        grid=(grid,),
        in_specs=[
            pl.BlockSpec((blk, D_NODE), lambda i: (i, 0)),
            pl.BlockSpec((D_NODE, 16), lambda i: (0, 0)),
            pl.BlockSpec((D_NODE, 16), lambda i: (0, 0)),
        ],
        out_specs=[
            pl.BlockSpec((blk, 16), lambda i: (i, 0)),
            pl.BlockSpec((blk, 16), lambda i: (i, 0)),
        ],
        out_shape=[
            jax.ShapeDtypeStruct((N_NODES, 16), jnp.float32),
            jax.ShapeDtypeStruct((N_NODES, 16), jnp.float32),
        ],
    )(x, ws_pad, wd_pad)


def _edge_proj_body(et_ref, wt_ref, bt_ref, z_ref):
    z = lax.dot_general(
        wt_ref[...], et_ref[...], (((1,), (0,)), ((), ())),
        preferred_element_type=jnp.float32, precision=lax.Precision.HIGHEST)
    z_ref[...] = z + bt_ref[...]


def _edge_proj_t(et, wet_pad, b7t_pad):
    blk = 12800
    grid = N_EDGES // blk
    return pl.pallas_call(
        _edge_proj_body,
        grid=(grid,),
        in_specs=[
            pl.BlockSpec((D_EDGE, blk), lambda i: (0, i)),
            pl.BlockSpec((8, D_EDGE), lambda i: (0, 0)),
            pl.BlockSpec((8, 1), lambda i: (0, 0)),
        ],
        out_specs=pl.BlockSpec((8, blk), lambda i: (0, i)),
        out_shape=jax.ShapeDtypeStruct((8, N_EDGES), jnp.float32),
    )(et, wet_pad, b7t_pad)


def _sc_body(zs_hbm, zd_hbm, zet_hbm, src_hbm, dst_hbm, wtab_hbm,
             out0_hbm, out1_hbm,
             idx_s, idx_d, buf_a, buf_b, ze_buf, out_buf, wbuf,
             sem_g, sem_o):
    wid = lax.axis_index("s") * NC + lax.axis_index("c")
    pltpu.sync_copy(wtab_hbm, wbuf)

    iota16 = lax.iota(jnp.int32, 16)
    col_idx = [jnp.full((16,), k, jnp.int32) for k in range(6)]

    def start_fetch(j, par):
        gid = wid * CHUNKS_PER_W + j
        r0 = gid * CHUNK
        pltpu.sync_copy(src_hbm.at[pl.ds(gid * SUBS, SUBS)], idx_s.at[par])
        pltpu.sync_copy(dst_hbm.at[pl.ds(gid * SUBS, SUBS)], idx_d.at[par])
        for k in range(SUBS):
            pltpu.async_copy(
                zs_hbm.at[idx_s.at[par].at[k]],
                buf_a.at[par].at[pl.ds(k * SUB, SUB)], sem_g.at[par])
            pltpu.async_copy(
                zd_hbm.at[idx_d.at[par].at[k]],
                buf_b.at[par].at[pl.ds(k * SUB, SUB)], sem_g.at[par])
        pltpu.async_copy(
            zet_hbm.at[:, pl.ds(r0, CHUNK)], ze_buf.at[par], sem_g.at[par])

    def drain_fetch(par):
        for k in range(SUBS):
            pltpu.make_async_copy(
                zs_hbm.at[idx_s.at[par].at[k]],
                buf_a.at[par].at[pl.ds(k * SUB, SUB)], sem_g.at[par]).wait()
            pltpu.make_async_copy(
                zd_hbm.at[idx_d.at[par].at[k]],
                buf_b.at[par].at[pl.ds(k * SUB, SUB)], sem_g.at[par]).wait()
        pltpu.make_async_copy(
            zet_hbm.at[:, pl.ds(0, CHUNK)], ze_buf.at[par],
            sem_g.at[par]).wait()

    def drain_out(j, par):
        r0 = (wid * CHUNKS_PER_W + j) * CHUNK
        pltpu.make_async_copy(
            out_buf.at[par].at[0], out0_hbm.at[pl.ds(r0, CHUNK)],
            sem_o.at[par]).wait()
        pltpu.make_async_copy(
            out_buf.at[par].at[1], out1_hbm.at[pl.ds(r0, CHUNK)],
            sem_o.at[par]).wait()

    start_fetch(0, 0)

    def chunk_body(j, carry):
        par = lax.rem(j, 2)
        nxt = lax.rem(j + 1, 2)
        gid = wid * CHUNKS_PER_W + j
        r0 = gid * CHUNK

        @pl.when(j + 1 < CHUNKS_PER_W)
        def _():
            start_fetch(j + 1, nxt)

        drain_fetch(par)

        bfa = buf_a.at[par]
        bfb = buf_b.at[par]
        zeb = ze_buf.at[par]
        obf = out_buf.at[par]

        def group_body(g, gcarry):
            rows = g * 16 + iota16
            g16 = g * 16
            h0 = []
            for k in range(6):
                a = plsc.load_gather(bfa, [rows, col_idx[k]])
                b = plsc.load_gather(bfb, [rows, col_idx[k]])
                z = zeb[k, pl.ds(g16, 16)]
                h0.append(_lrelu(a + b + z))
            h1 = []
            for jj in range(12):
                acc = wbuf[_B8_OFF + jj]
                for k in range(6):
                    acc = acc + h0[k] * wbuf[_W8_OFF + k * 12 + jj]
                h1.append(_lrelu(acc))
            h2 = []
            for jj in range(6):
                acc = wbuf[_B81_OFF + jj]
                for k in range(12):
                    acc = acc + h1[k] * wbuf[_W81_OFF + k * 6 + jj]
                h2.append(_lrelu(acc))
            o = []
            for jj in range(2):
                acc = wbuf[_B9_OFF + jj]
                for k in range(6):
                    acc = acc + h2[k] * wbuf[_W9_OFF + k * 2 + jj]
                o.append(acc)
            m = jnp.maximum(o[0], o[1])
            e0 = jnp.exp(o[0] - m)
            e1 = jnp.exp(o[1] - m)
            inv = 1.0 / (e0 + e1)
            obf[0, pl.ds(g16, 16)] = e0 * inv
            obf[1, pl.ds(g16, 16)] = e1 * inv
            return gcarry

        lax.fori_loop(0, GROUPS, group_body, 0)

        @pl.when(j >= 2)
        def _():
            drain_out(j - 2, par)

        pltpu.async_copy(obf.at[0], out0_hbm.at[pl.ds(r0, CHUNK)],
                         sem_o.at[par])
        pltpu.async_copy(obf.at[1], out1_hbm.at[pl.ds(r0, CHUNK)],
                         sem_o.at[par])
        return carry

    lax.fori_loop(0, CHUNKS_PER_W, chunk_body, 0)
    drain_out(CHUNKS_PER_W - 2, lax.rem(CHUNKS_PER_W - 2, 2))
    drain_out(CHUNKS_PER_W - 1, lax.rem(CHUNKS_PER_W - 1, 2))


def _sc_edge_mlp(zs, zd, zet, src2d, dst2d, wtab):
    mesh = plsc.VectorSubcoreMesh(core_axis_name="c", subcore_axis_name="s")
    fn = functools.partial(
        pl.kernel,
        out_type=[
            jax.ShapeDtypeStruct((N_EDGES,), jnp.float32),
            jax.ShapeDtypeStruct((N_EDGES,), jnp.float32),
        ],
        mesh=mesh,
        compiler_params=pltpu.CompilerParams(
            needs_layout_passes=False, use_tc_tiling_on_sc=False),
        scratch_types=[
            pltpu.VMEM((2, SUBS, SUB), jnp.int32),
            pltpu.VMEM((2, SUBS, SUB), jnp.int32),
            pltpu.VMEM((2, CHUNK, 16), jnp.float32),
            pltpu.VMEM((2, CHUNK, 16), jnp.float32),
            pltpu.VMEM((2, 8, CHUNK), jnp.float32),
            pltpu.VMEM((2, 2, CHUNK), jnp.float32),
            pltpu.VMEM((_WTAB_ROWS, 16), jnp.float32),
            pltpu.SemaphoreType.DMA((2,)),
            pltpu.SemaphoreType.DMA((2,)),
        ],
    )(_sc_body)
    return fn(zs, zd, zet, src2d, dst2d, wtab)


def kernel(x, e, edge_index, W7, b7, W8, b8, W81, b81, W9, b9):
    src2d = edge_index[0].astype(jnp.int32).reshape(N_EDGES // SUB, SUB)
    dst2d = edge_index[1].astype(jnp.int32).reshape(N_EDGES // SUB, SUB)

    ws_pad = jnp.zeros((D_NODE, 16), jnp.float32).at[:, :6].set(W7[:D_NODE])
    wd_pad = jnp.zeros((D_NODE, 16), jnp.float32).at[:, :6].set(
        W7[D_NODE:2 * D_NODE])
    wet_pad = jnp.zeros((8, D_EDGE), jnp.float32).at[:6, :].set(
        W7[2 * D_NODE:].T)
    b7t_pad = jnp.zeros((8, 1), jnp.float32).at[:6, 0].set(b7)

    zs, zd = _node_proj(x, ws_pad, wd_pad)
    zet = _edge_proj_t(e.T, wet_pad, b7t_pad)

    wtab = jnp.concatenate([
        W8.reshape(72), b8, W81.reshape(72), b81, W9.reshape(12), b9])
    wtab = jnp.broadcast_to(wtab[:, None], (_WTAB_ROWS, 16))

    p0, p1 = _sc_edge_mlp(zs, zd, zet, src2d, dst2d, wtab)
    return jnp.stack([p0, p1], axis=1)
